# baseline (speedup 1.0000x reference)
import functools

import jax
import jax.numpy as jnp
from jax import lax
from jax.experimental import pallas as pl
from jax.experimental.pallas import tpu as pltpu
from jax.experimental.pallas import tpu_sc as plsc

_LANE = 128
_K = 2
_PAIR = 2 * _K


@functools.lru_cache(maxsize=None)
def _make_lookup(num_rows: int, depth: int, vocab: int):
    info = plsc.get_sparse_core_info()
    nc, ns = info.num_cores, info.num_subcores
    nw = nc * ns
    assert num_rows % (nw * 2 * _PAIR) == 0
    rows_per_w = num_rows // nw
    pairs = rows_per_w // _PAIR
    supers = pairs // 2

    mesh = plsc.VectorSubcoreMesh(core_axis_name="c", subcore_axis_name="s")

    @functools.partial(
        pl.kernel,
        mesh=mesh,
        out_type=jax.ShapeDtypeStruct((num_rows * _LANE, depth), jnp.float32),
        scratch_types=[
            pltpu.VMEM((2, _PAIR, _LANE), jnp.int32),
            pltpu.VMEM((2, _K * _LANE, depth), jnp.float32),
            pltpu.VMEM_SHARED((vocab, depth), jnp.float32),
            pltpu.SemaphoreType.DMA,
            pltpu.SemaphoreType.DMA,
            pltpu.SemaphoreType.DMA,
            pltpu.SemaphoreType.DMA,
            pltpu.SemaphoreType.DMA,
            pltpu.SemaphoreType.DMA,
        ],
    )
    def lookup(table_hbm, idx_hbm, out_hbm, idx_v, rows_v, table_sh,
               sem_i0, sem_i1, sem_g0, sem_g1, sem_o0, sem_o1):
        sem_i = (sem_i0, sem_i1)
        sem_g = (sem_g0, sem_g1)
        sem_o = (sem_o0, sem_o1)
        sid = lax.axis_index("s")
        wid = sid * nc + lax.axis_index("c")
        wbase = wid * rows_per_w

        @pl.when(sid == 0)
        def _():
            pltpu.sync_copy(table_hbm, table_sh)

        plsc.subcore_barrier()

        def idx_cp(par, p):
            return pltpu.make_async_copy(
                idx_hbm.at[pl.ds(wbase + p * _PAIR, _PAIR)],
                idx_v.at[par],
                sem_i[par],
            )

        def fire_gather(b, par, p):
            for j in range(_K):
                pltpu.async_copy(
                    table_sh.at[idx_v.at[par].at[b * _K + j]],
                    rows_v.at[b].at[pl.ds(j * _LANE, _LANE)],
                    sem_g[b],
                )

        def drain_gather(b, par):
            for j in range(_K):
                pltpu.make_async_copy(
                    table_sh.at[idx_v.at[par].at[b * _K + j]],
                    rows_v.at[b].at[pl.ds(j * _LANE, _LANE)],
                    sem_g[b],
                ).wait()

        def out_cp(b, p):
            g = wbase + p * _PAIR + b * _K
            return pltpu.make_async_copy(
                rows_v.at[b],
                out_hbm.at[pl.ds(g * _LANE, _K * _LANE)],
                sem_o[b],
            )

        idx_cp(0, 0).start()
        idx_cp(0, 0).wait()
        idx_cp(1, 1).start()
        fire_gather(0, 0, 0)
        fire_gather(1, 0, 0)

        def pair_step(par, p, prefetch):
            drain_gather(0, par)
            out_cp(0, p).start()
            drain_gather(1, par)
            out_cp(1, p).start()
            if prefetch:
                idx_cp(par, p + 2).start()
            idx_cp(1 - par, p + 1).wait()
            out_cp(0, p).wait()
            fire_gather(0, 1 - par, p + 1)
            out_cp(1, p).wait()
            fire_gather(1, 1 - par, p + 1)

        def body(v, carry):
            p = 2 * v
            pair_step(0, p, True)
            pair_step(1, p + 1, True)
            return carry

        lax.fori_loop(0, supers - 1, body, 0)

        pair_step(0, pairs - 2, False)
        drain_gather(0, 1)
        out_cp(0, pairs - 1).start()
        drain_gather(1, 1)
        out_cp(1, pairs - 1).start()
        out_cp(0, pairs - 1).wait()
        out_cp(1, pairs - 1).wait()

    return lookup


def kernel(atomic_numbers, table):
    b, s = atomic_numbers.shape
    vocab, depth = table.shape
    idx = atomic_numbers.reshape(-1).astype(jnp.int32).reshape(-1, _LANE)
    out = _make_lookup(idx.shape[0], depth, vocab)(table, idx)
    return out.reshape(b, s, depth)


# speedup vs baseline: 1.4641x; 1.4641x over previous
import functools

import jax
import jax.numpy as jnp
from jax import lax
from jax.experimental import pallas as pl
from jax.experimental.pallas import tpu as pltpu
from jax.experimental.pallas import tpu_sc as plsc

_LANE = 128
_K = 2
_NB = 3


@functools.lru_cache(maxsize=None)
def _make_lookup(num_rows: int, depth: int, vocab: int):
    info = plsc.get_sparse_core_info()
    nc, ns = info.num_cores, info.num_subcores
    nw = nc * ns
    rows_per_w = num_rows // nw
    iters = rows_per_w // _K
    assert num_rows % (nw * _K) == 0
    assert (iters - 4) % _NB == 0
    body_reps = (iters - 4) // _NB

    mesh = plsc.VectorSubcoreMesh(core_axis_name="c", subcore_axis_name="s")

    @functools.partial(
        pl.kernel,
        mesh=mesh,
        out_type=jax.ShapeDtypeStruct((num_rows * _LANE, depth), jnp.float32),
        scratch_types=[
            pltpu.VMEM((_NB, _K, _LANE), jnp.int32),
            pltpu.VMEM((_NB, _K * _LANE, depth), jnp.float32),
            pltpu.VMEM_SHARED((vocab, depth), jnp.float32),
            pltpu.SemaphoreType.DMA,
            pltpu.SemaphoreType.DMA,
            pltpu.SemaphoreType.DMA,
            pltpu.SemaphoreType.DMA,
            pltpu.SemaphoreType.DMA,
            pltpu.SemaphoreType.DMA,
        ],
    )
    def lookup(table_hbm, idx_hbm, out_hbm, idx_v, rows_v, table_sh,
               sem_g0, sem_g1, sem_g2, sem_o0, sem_o1, sem_o2):
        sem_g = (sem_g0, sem_g1, sem_g2)
        sem_o = (sem_o0, sem_o1, sem_o2)
        sid = lax.axis_index("s")
        wid = sid * nc + lax.axis_index("c")
        wbase = wid * rows_per_w

        @pl.when(sid == 0)
        def _():
            pltpu.sync_copy(table_hbm, table_sh)

        plsc.subcore_barrier()

        def fire_gather(r, t):
            pltpu.sync_copy(
                idx_hbm.at[pl.ds(wbase + t * _K, _K)], idx_v.at[r])
            for j in range(_K):
                pltpu.async_copy(
                    table_sh.at[idx_v.at[r].at[j]],
                    rows_v.at[r].at[pl.ds(j * _LANE, _LANE)],
                    sem_g[r],
                )

        def drain_gather(r):
            for j in range(_K):
                pltpu.make_async_copy(
                    table_sh.at[idx_v.at[r].at[j]],
                    rows_v.at[r].at[pl.ds(j * _LANE, _LANE)],
                    sem_g[r],
                ).wait()

        def out_cp(r, t):
            g = wbase + t * _K
            return pltpu.make_async_copy(
                rows_v.at[r],
                out_hbm.at[pl.ds(g * _LANE, _K * _LANE)],
                sem_o[r],
            )

        def step(t_ref, t, last=False):
            r = t % _NB
            if t >= 2:
                out_cp((t + 1) % _NB, t_ref - 2).wait()
            if not last:
                fire_gather((t + 1) % _NB, t_ref + 1)
            drain_gather(r)
            out_cp(r, t_ref).start()

        fire_gather(0, 0)
        step(0, 0)
        step(1, 1)

        def body(v, carry):
            t0 = 2 + v * _NB
            step(t0, 2)
            step(t0 + 1, 3)
            step(t0 + 2, 4)
            return carry

        lax.fori_loop(0, body_reps, body, 0)

        t = iters - 2
        step(t, t % _NB + _NB)
        step(t + 1, (t + 1) % _NB + _NB, last=True)
        out_cp((iters - 2) % _NB, iters - 2).wait()
        out_cp((iters - 1) % _NB, iters - 1).wait()

    return lookup


def kernel(atomic_numbers, table):
    b, s = atomic_numbers.shape
    vocab, depth = table.shape
    idx = atomic_numbers.reshape(-1).astype(jnp.int32).reshape(-1, _LANE)
    out = _make_lookup(idx.shape[0], depth, vocab)(table, idx)
    return out.reshape(b, s, depth)
